# Initial kernel scaffold; baseline (speedup 1.0000x reference)
import jax
import jax.numpy as jnp
from jax.experimental import pallas as pl


def kernel(cls_heads, reg_heads, batch_anchors):
    raise NotImplementedError("write your pallas kernel here")


# speedup vs baseline: 29.7892x; 29.7892x over previous
import jax
import jax.numpy as jnp
from jax.experimental import pallas as pl

IMAGE_W = 640
IMAGE_H = 640
TOP_N = 1000
MIN_SCORE = 0.05
NMS_THR = 0.5
MAX_DET = 100

L = 5
B = 4
N = 20000
C = 80
NPAD = 20480
RPL = NPAD // 128
ROWS = L * RPL


BLKA = 2000


def _reduce_body(cls_ref, s_ref, c_ref):
    x = cls_ref[0]
    m = jnp.max(x, axis=-1, keepdims=True)
    lane = jax.lax.broadcasted_iota(jnp.int32, (BLKA, C), 1)
    c = jnp.min(jnp.where(x == m, lane, C), axis=-1, keepdims=True)
    s_ref[0] = m
    c_ref[0] = c.astype(jnp.float32)


def _scores_classes(cls_heads):
    cls_r = cls_heads.reshape(L * B, N, C)
    return pl.pallas_call(
        _reduce_body,
        grid=(L * B, N // BLKA),
        in_specs=[pl.BlockSpec((1, BLKA, C), lambda i, j: (i, j, 0))],
        out_specs=[
            pl.BlockSpec((1, BLKA, 1), lambda i, j: (i, j, 0)),
            pl.BlockSpec((1, BLKA, 1), lambda i, j: (i, j, 0)),
        ],
        out_shape=[
            jax.ShapeDtypeStruct((L * B, N, 1), jnp.float32),
            jax.ShapeDtypeStruct((L * B, N, 1), jnp.float32),
        ],
    )(cls_r)


def _nms_body(sp_ref, bits_ref, cls_ref, reg_ref, anc_ref,
              os_ref, oc_ref, ob_ref):
    sp = sp_ref[0]
    bits = bits_ref[0]
    clsf = cls_ref[0]

    il = jax.lax.broadcasted_iota(jnp.int32, (RPL, 128), 0) * 128 + \
        jax.lax.broadcasted_iota(jnp.int32, (RPL, 128), 1)
    masked = []
    for l in range(L):
        bl = jax.lax.slice(bits, (l * RPL, 0), ((l + 1) * RPL, 128))
        sl = jax.lax.slice(sp, (l * RPL, 0), ((l + 1) * RPL, 128))

        def bs_body(_, lohi, bl=bl):
            lo, hi = lohi
            mid = (lo + hi) // 2
            cnt = jnp.sum((bl >= mid).astype(jnp.int32))
            big = cnt >= TOP_N
            return jnp.where(big, mid, lo), jnp.where(big, hi, mid)

        lo, _ = jax.lax.fori_loop(
            0, 30, bs_body,
            (jnp.int32(0), jnp.int32(1 << 30)))
        vk = lo
        cnt_gt = jnp.sum((bl > vk).astype(jnp.int32))
        m_need = TOP_N - cnt_gt
        eq = bl == vk

        def ib_body(_, lohi, eq=eq, m_need=m_need):
            lo2, hi2 = lohi
            mid2 = (lo2 + hi2) // 2
            f = jnp.sum((eq & (il < mid2)).astype(jnp.int32))
            ge = f >= m_need
            return jnp.where(ge, lo2, mid2), jnp.where(ge, mid2, hi2)

        _, t_hi = jax.lax.fori_loop(
            0, 16, ib_body,
            (jnp.int32(0), jnp.int32(1 << 15)))
        sel = (bl > vk) | (eq & (il < t_hi))
        masked.append(jnp.where(sel & (sl > MIN_SCORE), sl, -1.0))
    ms = jnp.concatenate(masked, axis=0)

    a0 = anc_ref[0, 0]
    a1 = anc_ref[0, 1]
    a2 = anc_ref[0, 2]
    a3 = anc_ref[0, 3]
    r0 = reg_ref[0, 0] * jnp.float32(0.1)
    r1 = reg_ref[0, 1] * jnp.float32(0.1)
    r2 = reg_ref[0, 2] * jnp.float32(0.2)
    r3 = reg_ref[0, 3] * jnp.float32(0.2)
    wh_x = a2 - a0
    wh_y = a3 - a1
    ctr_x = a0 + jnp.float32(0.5) * wh_x
    ctr_y = a1 + jnp.float32(0.5) * wh_y
    pw = jnp.exp(r2) * wh_x
    ph = jnp.exp(r3) * wh_y
    pcx = r0 * wh_x + ctr_x
    pcy = r1 * wh_y + ctr_y
    x1 = jnp.maximum((pcx - jnp.float32(0.5) * pw).astype(jnp.int32), 0)
    y1 = jnp.maximum((pcy - jnp.float32(0.5) * ph).astype(jnp.int32), 0)
    x2 = jnp.minimum((pcx + jnp.float32(0.5) * pw).astype(jnp.int32), IMAGE_W)
    y2 = jnp.minimum((pcy + jnp.float32(0.5) * ph).astype(jnp.int32), IMAGE_H)
    x1 = x1.astype(jnp.float32)
    y1 = y1.astype(jnp.float32)
    x2 = x2.astype(jnp.float32)
    y2 = y2.astype(jnp.float32)
    areas = (x2 - x1) * (y2 - y1)

    lin = jax.lax.broadcasted_iota(jnp.int32, (ROWS, 128), 0) * 128 + \
        jax.lax.broadcasted_iota(jnp.int32, (ROWS, 128), 1)
    lane = jax.lax.broadcasted_iota(jnp.int32, (1, 128), 1)
    neg = jnp.full((1, 128), -1.0, dtype=jnp.float32)

    def nms_body(i, carry):
        msc, osv, ocv, ox1, oy1, ox2, oy2 = carry
        m = jnp.max(msc)
        keep = m > 0.0
        sel_idx = jnp.min(jnp.where(msc == m, lin, 1 << 30))
        onehot = lin == sel_idx
        cx1 = jnp.sum(jnp.where(onehot, x1, 0.0))
        cy1 = jnp.sum(jnp.where(onehot, y1, 0.0))
        cx2 = jnp.sum(jnp.where(onehot, x2, 0.0))
        cy2 = jnp.sum(jnp.where(onehot, y2, 0.0))
        car = jnp.sum(jnp.where(onehot, areas, 0.0))
        ccl = jnp.sum(jnp.where(onehot, clsf, 0.0))
        upd = (lane == i) & keep
        osv = jnp.where(upd, m, osv)
        ocv = jnp.where(upd, ccl, ocv)
        ox1 = jnp.where(upd, cx1, ox1)
        oy1 = jnp.where(upd, cy1, oy1)
        ox2 = jnp.where(upd, cx2, ox2)
        oy2 = jnp.where(upd, cy2, oy2)
        tlx = jnp.maximum(x1, cx1)
        tly = jnp.maximum(y1, cy1)
        brx = jnp.minimum(x2, cx2)
        bry = jnp.minimum(y2, cy2)
        szx = jnp.maximum(brx - tlx, 0.0)
        szy = jnp.maximum(bry - tly, 0.0)
        ov = szx * szy
        union = jnp.maximum(car + areas - ov, jnp.float32(0.0001))
        iou = ov / union
        remove = keep & ((iou >= NMS_THR) | onehot)
        msc = jnp.where(remove, -1.0, msc)
        return msc, osv, ocv, ox1, oy1, ox2, oy2

    _, osv, ocv, ox1, oy1, ox2, oy2 = jax.lax.fori_loop(
        0, MAX_DET, nms_body, (ms, neg, neg, neg, neg, neg, neg))
    os_ref[0] = osv
    oc_ref[0] = ocv
    ob_ref[0] = jnp.concatenate([ox1, oy1, ox2, oy2], axis=0)


def _to_rows(x, pad_value):
    xp = jnp.pad(x, ((0, 0), (0, 0), (0, NPAD - N)),
                 constant_values=pad_value)
    return xp.reshape(L, B, RPL, 128).transpose(1, 0, 2, 3).reshape(
        B, ROWS, 128)


def _coords_to_rows(x):
    xp = jnp.pad(x, ((0, 0), (0, 0), (0, NPAD - N), (0, 0)))
    return xp.reshape(L, B, RPL, 128, 4).transpose(1, 4, 0, 2, 3).reshape(
        B, 4, ROWS, 128)


def kernel(cls_heads, reg_heads, batch_anchors):
    s, c = _scores_classes(cls_heads)
    sp = _to_rows(s.reshape(L, B, N), -1.0)
    bits = jax.lax.bitcast_convert_type(sp, jnp.int32)
    clsf = _to_rows(c.reshape(L, B, N), -1.0)
    reg_t = _coords_to_rows(reg_heads)
    anc_t = _coords_to_rows(batch_anchors)

    out_s, out_c, out_b = pl.pallas_call(
        _nms_body,
        grid=(B,),
        in_specs=[
            pl.BlockSpec((1, ROWS, 128), lambda i: (i, 0, 0)),
            pl.BlockSpec((1, ROWS, 128), lambda i: (i, 0, 0)),
            pl.BlockSpec((1, ROWS, 128), lambda i: (i, 0, 0)),
            pl.BlockSpec((1, 4, ROWS, 128), lambda i: (i, 0, 0, 0)),
            pl.BlockSpec((1, 4, ROWS, 128), lambda i: (i, 0, 0, 0)),
        ],
        out_specs=[
            pl.BlockSpec((1, 1, 128), lambda i: (i, 0, 0)),
            pl.BlockSpec((1, 1, 128), lambda i: (i, 0, 0)),
            pl.BlockSpec((1, 4, 128), lambda i: (i, 0, 0)),
        ],
        out_shape=[
            jax.ShapeDtypeStruct((B, 1, 128), jnp.float32),
            jax.ShapeDtypeStruct((B, 1, 128), jnp.float32),
            jax.ShapeDtypeStruct((B, 4, 128), jnp.float32),
        ],
    )(sp, bits, clsf, reg_t, anc_t)

    scores = out_s[:, 0, :MAX_DET]
    classes = out_c[:, 0, :MAX_DET]
    boxes = out_b.transpose(0, 2, 1)[:, :MAX_DET, :]
    return scores, classes, boxes


# speedup vs baseline: 31.5151x; 1.0579x over previous
import jax
import jax.numpy as jnp
from jax.experimental import pallas as pl

IMAGE_W = 640
IMAGE_H = 640
TOP_N = 1000
MIN_SCORE = 0.05
NMS_THR = 0.5
MAX_DET = 100

L = 5
B = 4
N = 20000
C = 80
NPAD = 20480
RPL = NPAD // 128
ROWS = L * RPL


BLKA = 2000


def _reduce_body(cls_ref, s_ref, c_ref):
    x = cls_ref[0]
    m = jnp.max(x, axis=-1, keepdims=True)
    lane = jax.lax.broadcasted_iota(jnp.int32, (BLKA, C), 1)
    c = jnp.min(jnp.where(x == m, lane, C), axis=-1, keepdims=True)
    s_ref[0] = m
    c_ref[0] = c.astype(jnp.float32)


def _scores_classes(cls_heads):
    cls_r = cls_heads.reshape(L * B, N, C)
    return pl.pallas_call(
        _reduce_body,
        grid=(L * B, N // BLKA),
        in_specs=[pl.BlockSpec((1, BLKA, C), lambda i, j: (i, j, 0))],
        out_specs=[
            pl.BlockSpec((1, BLKA, 1), lambda i, j: (i, j, 0)),
            pl.BlockSpec((1, BLKA, 1), lambda i, j: (i, j, 0)),
        ],
        out_shape=[
            jax.ShapeDtypeStruct((L * B, N, 1), jnp.float32),
            jax.ShapeDtypeStruct((L * B, N, 1), jnp.float32),
        ],
    )(cls_r)


def _nms_body(sp_ref, bits_ref, cls_ref, reg_ref, anc_ref,
              os_ref, oc_ref, ob_ref):
    sp = sp_ref[0]
    bits = bits_ref[0]
    clsf = cls_ref[0]

    il = jax.lax.broadcasted_iota(jnp.int32, (RPL, 128), 0) * 128 + \
        jax.lax.broadcasted_iota(jnp.int32, (RPL, 128), 1)
    masked = []
    for l in range(L):
        bl = jax.lax.slice(bits, (l * RPL, 0), ((l + 1) * RPL, 128))
        sl = jax.lax.slice(sp, (l * RPL, 0), ((l + 1) * RPL, 128))

        def bs_body(_, lohi, bl=bl):
            lo, hi = lohi
            mid = (lo + hi) // 2
            cnt = jnp.sum((bl >= mid).astype(jnp.int32))
            big = cnt >= TOP_N
            return jnp.where(big, mid, lo), jnp.where(big, hi, mid)

        lo, _ = jax.lax.fori_loop(
            0, 30, bs_body,
            (jnp.int32(0), jnp.int32(1 << 30)))
        vk = lo
        cnt_gt = jnp.sum((bl > vk).astype(jnp.int32))
        m_need = TOP_N - cnt_gt
        eq = bl == vk

        def ib_body(_, lohi, eq=eq, m_need=m_need):
            lo2, hi2 = lohi
            mid2 = (lo2 + hi2) // 2
            f = jnp.sum((eq & (il < mid2)).astype(jnp.int32))
            ge = f >= m_need
            return jnp.where(ge, lo2, mid2), jnp.where(ge, mid2, hi2)

        _, t_hi = jax.lax.fori_loop(
            0, 16, ib_body,
            (jnp.int32(0), jnp.int32(1 << 15)))
        sel = (bl > vk) | (eq & (il < t_hi))
        masked.append(jnp.where(sel & (sl > MIN_SCORE), sl, -1.0))
    ms = jnp.concatenate(masked, axis=0)

    a0 = anc_ref[0, 0]
    a1 = anc_ref[0, 1]
    a2 = anc_ref[0, 2]
    a3 = anc_ref[0, 3]
    r0 = reg_ref[0, 0] * jnp.float32(0.1)
    r1 = reg_ref[0, 1] * jnp.float32(0.1)
    r2 = reg_ref[0, 2] * jnp.float32(0.2)
    r3 = reg_ref[0, 3] * jnp.float32(0.2)
    wh_x = a2 - a0
    wh_y = a3 - a1
    ctr_x = a0 + jnp.float32(0.5) * wh_x
    ctr_y = a1 + jnp.float32(0.5) * wh_y
    pw = jnp.exp(r2) * wh_x
    ph = jnp.exp(r3) * wh_y
    pcx = r0 * wh_x + ctr_x
    pcy = r1 * wh_y + ctr_y
    x1 = jnp.maximum((pcx - jnp.float32(0.5) * pw).astype(jnp.int32), 0)
    y1 = jnp.maximum((pcy - jnp.float32(0.5) * ph).astype(jnp.int32), 0)
    x2 = jnp.minimum((pcx + jnp.float32(0.5) * pw).astype(jnp.int32), IMAGE_W)
    y2 = jnp.minimum((pcy + jnp.float32(0.5) * ph).astype(jnp.int32), IMAGE_H)
    x1 = x1.astype(jnp.float32)
    y1 = y1.astype(jnp.float32)
    x2 = x2.astype(jnp.float32)
    y2 = y2.astype(jnp.float32)
    areas = (x2 - x1) * (y2 - y1)

    lin = jax.lax.broadcasted_iota(jnp.int32, (ROWS, 128), 0) * 128 + \
        jax.lax.broadcasted_iota(jnp.int32, (ROWS, 128), 1)
    lane = jax.lax.broadcasted_iota(jnp.int32, (1, 128), 1)
    neg = jnp.full((1, 128), -1.0, dtype=jnp.float32)

    def nms_body(i, carry):
        msc, osv, ocv, ox1, oy1, ox2, oy2 = carry
        m = jnp.max(msc)
        keep = m > 0.0
        sel_idx = jnp.min(jnp.where(msc == m, lin, 1 << 30))
        onehot = lin == sel_idx
        cx1 = jnp.sum(jnp.where(onehot, x1, 0.0))
        cy1 = jnp.sum(jnp.where(onehot, y1, 0.0))
        cx2 = jnp.sum(jnp.where(onehot, x2, 0.0))
        cy2 = jnp.sum(jnp.where(onehot, y2, 0.0))
        car = jnp.sum(jnp.where(onehot, areas, 0.0))
        ccl = jnp.sum(jnp.where(onehot, clsf, 0.0))
        upd = (lane == i) & keep
        osv = jnp.where(upd, m, osv)
        ocv = jnp.where(upd, ccl, ocv)
        ox1 = jnp.where(upd, cx1, ox1)
        oy1 = jnp.where(upd, cy1, oy1)
        ox2 = jnp.where(upd, cx2, ox2)
        oy2 = jnp.where(upd, cy2, oy2)
        tlx = jnp.maximum(x1, cx1)
        tly = jnp.maximum(y1, cy1)
        brx = jnp.minimum(x2, cx2)
        bry = jnp.minimum(y2, cy2)
        szx = jnp.maximum(brx - tlx, 0.0)
        szy = jnp.maximum(bry - tly, 0.0)
        ov = szx * szy
        union = jnp.maximum(car + areas - ov, jnp.float32(0.0001))
        iou = ov / union
        remove = keep & ((iou >= NMS_THR) | onehot)
        msc = jnp.where(remove, -1.0, msc)
        return msc, osv, ocv, ox1, oy1, ox2, oy2

    _, osv, ocv, ox1, oy1, ox2, oy2 = jax.lax.fori_loop(
        0, MAX_DET, nms_body, (ms, neg, neg, neg, neg, neg, neg))
    os_ref[0] = osv
    oc_ref[0] = ocv
    ob_ref[0] = jnp.concatenate([ox1, oy1, ox2, oy2], axis=0)


def _to_rows(x, pad_value):
    xp = jnp.pad(x, ((0, 0), (0, 0), (0, NPAD - N)),
                 constant_values=pad_value)
    return xp.reshape(L, B, RPL, 128).transpose(1, 0, 2, 3).reshape(
        B, ROWS, 128)


def _coords_to_rows(x):
    return jnp.stack([_to_rows(x[..., c], 0.0) for c in range(4)], axis=1)


def kernel(cls_heads, reg_heads, batch_anchors):
    s, c = _scores_classes(cls_heads)
    sp = _to_rows(s.reshape(L, B, N), -1.0)
    bits = jax.lax.bitcast_convert_type(sp, jnp.int32)
    clsf = _to_rows(c.reshape(L, B, N), -1.0)
    reg_t = _coords_to_rows(reg_heads)
    anc_t = _coords_to_rows(batch_anchors)

    out_s, out_c, out_b = pl.pallas_call(
        _nms_body,
        grid=(B,),
        in_specs=[
            pl.BlockSpec((1, ROWS, 128), lambda i: (i, 0, 0)),
            pl.BlockSpec((1, ROWS, 128), lambda i: (i, 0, 0)),
            pl.BlockSpec((1, ROWS, 128), lambda i: (i, 0, 0)),
            pl.BlockSpec((1, 4, ROWS, 128), lambda i: (i, 0, 0, 0)),
            pl.BlockSpec((1, 4, ROWS, 128), lambda i: (i, 0, 0, 0)),
        ],
        out_specs=[
            pl.BlockSpec((1, 1, 128), lambda i: (i, 0, 0)),
            pl.BlockSpec((1, 1, 128), lambda i: (i, 0, 0)),
            pl.BlockSpec((1, 4, 128), lambda i: (i, 0, 0)),
        ],
        out_shape=[
            jax.ShapeDtypeStruct((B, 1, 128), jnp.float32),
            jax.ShapeDtypeStruct((B, 1, 128), jnp.float32),
            jax.ShapeDtypeStruct((B, 4, 128), jnp.float32),
        ],
    )(sp, bits, clsf, reg_t, anc_t)

    scores = out_s[:, 0, :MAX_DET]
    classes = out_c[:, 0, :MAX_DET]
    boxes = out_b.transpose(0, 2, 1)[:, :MAX_DET, :]
    return scores, classes, boxes
